# baseline (speedup 1.0000x reference)
import functools

import jax
import jax.numpy as jnp
from jax import lax
from jax.experimental import pallas as pl
from jax.experimental.pallas import tpu as pltpu
from jax.experimental.pallas import tpu_sc as plsc

D = 64
NC, NS = 2, 16
NW = NC * NS
CHUNK = 128


@functools.lru_cache(maxsize=None)
def _make(B):
    assert B % (NW * CHUNK) == 0
    b_per_w = B // NW
    n_chunks = b_per_w // CHUNK
    mesh = plsc.VectorSubcoreMesh(core_axis_name="c", subcore_axis_name="s")

    @functools.partial(
        pl.kernel,
        mesh=mesh,
        out_type=jax.ShapeDtypeStruct((B, D), jnp.float32),
        compiler_params=pltpu.CompilerParams(use_tc_tiling_on_sc=False),
        scratch_types=[
            pltpu.VMEM((CHUNK,), jnp.int32),
            pltpu.VMEM((CHUNK, D), jnp.float32),
            pltpu.SemaphoreType.DMA,
        ],
    )
    def k(idx_hbm, table_hbm, out_hbm, idx_v, rows_v, sem):
        wid = lax.axis_index("s") * NC + lax.axis_index("c")
        base = wid * b_per_w

        def body(i, carry):
            start = base + i * CHUNK
            pltpu.sync_copy(idx_hbm.at[pl.ds(start, CHUNK)], idx_v)
            pltpu.async_copy(table_hbm.at[idx_v], rows_v, sem).wait()
            pltpu.sync_copy(rows_v, out_hbm.at[pl.ds(start, CHUNK)])
            return carry

        lax.fori_loop(0, n_chunks, body, 0)

    return k


@jax.jit
def kernel(x, table):
    r, c = x.shape
    B = r * c
    x_flat = x.reshape(B).astype(jnp.int32)
    out = _make(B)(x_flat, table)
    return out.reshape(r, c, D)


# speedup vs baseline: 1.1102x; 1.1102x over previous
import functools

import jax
import jax.numpy as jnp
from jax import lax
from jax.experimental import pallas as pl
from jax.experimental.pallas import tpu as pltpu
from jax.experimental.pallas import tpu_sc as plsc

D = 64
NC, NS = 2, 16
NW = NC * NS
CHUNK = 128
NBUF = 4


@functools.lru_cache(maxsize=None)
def _make(B):
    assert B % (NW * CHUNK * NBUF) == 0
    b_per_w = B // NW
    n_chunks = b_per_w // CHUNK
    mesh = plsc.VectorSubcoreMesh(core_axis_name="c", subcore_axis_name="s")

    @functools.partial(
        pl.kernel,
        mesh=mesh,
        out_type=jax.ShapeDtypeStruct((B, D), jnp.float32),
        compiler_params=pltpu.CompilerParams(use_tc_tiling_on_sc=False),
        scratch_types=[
            pltpu.VMEM((b_per_w,), jnp.int32),
            pltpu.VMEM((NBUF, CHUNK, D), jnp.float32),
            pltpu.SemaphoreType.DMA((NBUF,)),
        ],
    )
    def k(idx_hbm, table_hbm, out_hbm, idx_v, bufs, sems):
        wid = lax.axis_index("s") * NC + lax.axis_index("c")
        base = wid * b_per_w
        pltpu.sync_copy(idx_hbm.at[pl.ds(base, b_per_w)], idx_v)

        def gather(i, b):
            pltpu.make_async_copy(
                table_hbm.at[idx_v.at[pl.ds(i * CHUNK, CHUNK)]],
                bufs.at[b],
                sems.at[b],
            ).start()

        for b in range(NBUF):
            gather(b, b)

        def body(g, carry):
            c = g * NBUF
            for b in range(NBUF):
                i = c + b
                pltpu.make_async_copy(
                    table_hbm.at[idx_v.at[pl.ds(0, CHUNK)]],
                    bufs.at[b],
                    sems.at[b],
                ).wait()
                pltpu.sync_copy(
                    bufs.at[b], out_hbm.at[pl.ds(base + i * CHUNK, CHUNK)]
                )
                nxt = i + NBUF

                @pl.when(nxt < n_chunks)
                def _():
                    gather(nxt, b)

            return carry

        lax.fori_loop(0, n_chunks // NBUF, body, 0)

    return k


@jax.jit
def kernel(x, table):
    r, c = x.shape
    B = r * c
    x_flat = x.reshape(B).astype(jnp.int32)
    out = _make(B)(x_flat, table)
    return out.reshape(r, c, D)
